# baseline (speedup 1.0000x reference)
import numpy as np
import jax
import jax.numpy as jnp
from jax import lax
from jax.experimental import pallas as pl
from jax.experimental.pallas import tpu as pltpu
from jax.experimental.pallas import tpu_sc as plsc

_B, _N = 4096, 128
_K = _N * (_N + 1) // 2
_TK = _K // 8
_NW = 32
_BPW = _B // _NW

_SEG_OFF = [i * (2 * _N + 1 - i) // 2 for i in range(_N + 1)]


def _row_of_k(k):
    import bisect
    return bisect.bisect_right(_SEG_OFF, k) - 1


_TAIL_TILES = []
for _t in (_TK - 2, _TK - 1):
    cols = []
    for _kr in range(8):
        _k = 8 * _t + _kr
        _i = _row_of_k(_k)
        _col = _i + (_k - _SEG_OFF[_i])
        cols.append(8 * (_i - 122) + (_col - 120))
    _TAIL_TILES.append((_t, cols))


def _sc_body(x_hbm, out_hbm, slab_a, slab_b, tail_v, stage_v, isem, osem):
    c = lax.axis_index("c")
    s = lax.axis_index("s")
    w = s * 2 + c
    bc = w * _BPW

    iota = lax.iota(jnp.int32, 16)

    def start_in(slab, i):
        pltpu.async_copy(x_hbm.at[pl.ds(bc, _BPW), i, :], slab, isem)

    def wait_in(slab, i):
        pltpu.make_async_copy(x_hbm.at[pl.ds(bc, _BPW), i, :], slab,
                              isem).wait()

    def drain_out():
        pltpu.make_async_copy(stage_v.at[0], out_hbm.at[0, w], osem).wait()

    def ring_pre(g):
        @pl.when(g >= 8)
        def _():
            drain_out()

    def ring_post(t, g):
        pltpu.async_copy(stage_v.at[g & 7], out_hbm.at[t, w], osem)
        return g + 1

    def gather_row(slab, par, kr, col):
        colv = jnp.full((16,), col, dtype=jnp.int32)
        for bs in range(8):
            br = iota + (bs * 16)
            stage_v[par, kr, pl.ds(bs * 16, 16)] = plsc.load_gather(
                slab, [br, colv])

    def full_strips(slab, i, off_i, off_n, g):
        t_lo = (off_i + 7) // 8
        t_hi = jnp.maximum(t_lo, off_n // 8)

        def body(t, g2):
            ring_pre(g2)
            for kr in range(8):
                gather_row(slab, g2 & 7, kr, i + (8 * t + kr - off_i))
            return ring_post(t, g2)

        return lax.fori_loop(t_lo, t_hi, body, g)

    def straddle(prev, cur, i, off_i, g):
        t0 = off_i // 8
        n1 = off_i - 8 * t0

        def do(g2):
            ring_pre(g2)
            par = g2 & 7

            def kprev(kr, u):
                gather_row(prev, par, kr, _N - n1 + kr)
                return u

            def kcur(kr, u):
                gather_row(cur, par, kr, i + kr - n1)
                return u

            lax.fori_loop(0, n1, kprev, 0)
            lax.fori_loop(n1, 8, kcur, 0)
            return ring_post(t0, g2)

        return lax.cond((n1 != 0) & (t0 <= _TK - 3) & (i > 0),
                        do, lambda g2: g2, g)

    start_in(slab_a, 0)

    def pair(ip, g):
        i0 = ip * 2
        i1 = i0 + 1
        off0 = i0 * (2 * _N + 1 - i0) // 2
        off1 = off0 + (_N - i0)
        off2 = off1 + (_N - i1)

        wait_in(slab_a, i0)
        g = straddle(slab_b, slab_a, i0, off0, g)
        start_in(slab_b, i1)
        g = full_strips(slab_a, i0, off0, off1, g)

        wait_in(slab_b, i1)
        g = straddle(slab_a, slab_b, i1, off1, g)

        @pl.when(i1 + 1 < _N)
        def _():
            start_in(slab_a, i1 + 1)

        g = full_strips(slab_b, i1, off1, off2, g)
        return g

    g = lax.fori_loop(0, _N // 2, pair, 0)

    for j in range(6):
        pltpu.sync_copy(x_hbm.at[pl.ds(bc, _BPW), 122 + j, pl.ds(120, 8)],
                        tail_v.at[:, pl.ds(8 * j, 8)])

    for t, cols in _TAIL_TILES:
        ring_pre(g)
        par_t = g & 7
        for kr in range(8):
            colv = jnp.full((16,), cols[kr], dtype=jnp.int32)
            for bs in range(8):
                br = iota + (bs * 16)
                stage_v[par_t, kr, pl.ds(bs * 16, 16)] = plsc.load_gather(
                    tail_v, [br, colv])
        g = ring_post(t, g)

    for _ in range(8):
        drain_out()


def kernel(input):
    mesh = plsc.VectorSubcoreMesh(core_axis_name="c", subcore_axis_name="s")
    f = pl.kernel(
        _sc_body,
        mesh=mesh,
        out_type=jax.ShapeDtypeStruct((_TK, _NW, 8, _BPW), jnp.float32),
        scratch_types=[
            pltpu.VMEM((_BPW, _N), jnp.float32),
            pltpu.VMEM((_BPW, _N), jnp.float32),
            pltpu.VMEM((_BPW, 48), jnp.float32),
            pltpu.VMEM((8, 8, _BPW), jnp.float32),
            pltpu.SemaphoreType.DMA,
            pltpu.SemaphoreType.DMA,
        ],
        compiler_params=pltpu.CompilerParams(
            use_tc_tiling_on_sc=False, needs_layout_passes=False
        ),
    )
    r4 = f(input)
    return r4.transpose(1, 3, 0, 2).reshape(_B, _K)


# speedup vs baseline: 1.2093x; 1.2093x over previous
import numpy as np
import jax
import jax.numpy as jnp
from jax import lax
from jax.experimental import pallas as pl
from jax.experimental.pallas import tpu as pltpu
from jax.experimental.pallas import tpu_sc as plsc

_B, _N = 4096, 128
_K = _N * (_N + 1) // 2
_TK = _K // 8
_NW = 32
_BPW = _B // _NW

_SEG_OFF = [i * (2 * _N + 1 - i) // 2 for i in range(_N + 1)]


def _row_of_k(k):
    import bisect
    return bisect.bisect_right(_SEG_OFF, k) - 1


_TAIL_TILES = []
for _t in (_TK - 2, _TK - 1):
    cols = []
    for _kr in range(8):
        _k = 8 * _t + _kr
        _i = _row_of_k(_k)
        _col = _i + (_k - _SEG_OFF[_i])
        cols.append(8 * (_i - 122) + (_col - 120))
    _TAIL_TILES.append((_t, cols))


def _sc_body(x_hbm, out_hbm, slab_a, slab_b, tail_v, stage_v, isem, osem):
    c = lax.axis_index("c")
    s = lax.axis_index("s")
    w = s * 2 + c
    bc = w * _BPW

    iota = lax.iota(jnp.int32, 16)

    def start_in(slab, i):
        pltpu.async_copy(x_hbm.at[pl.ds(bc, _BPW), i, :], slab, isem)

    def wait_in(slab, i):
        pltpu.make_async_copy(x_hbm.at[pl.ds(bc, _BPW), i, :], slab,
                              isem).wait()

    def drain_out():
        pltpu.make_async_copy(stage_v.at[0], out_hbm.at[0, w], osem).wait()

    def ring_pre(g):
        @pl.when(g >= 8)
        def _():
            drain_out()

    def ring_post(t, g):
        pltpu.async_copy(stage_v.at[g & 7], out_hbm.at[t, w], osem)
        return g + 1

    def gather_row(slab, par, kr, colv):
        vals = []
        for bs in range(8):
            br = iota + (bs * 16)
            vals.append(plsc.load_gather(slab, [br, colv]))
        for bs in range(8):
            stage_v[par, kr, pl.ds(bs * 16, 16)] = vals[bs]

    def full_strips(slab, i, off_i, off_n, g):
        t_lo = (off_i + 7) // 8
        t_hi = jnp.maximum(t_lo, off_n // 8)

        def body(t, g2):
            ring_pre(g2)
            colbase = jnp.full((16,), i + (8 * t - off_i), dtype=jnp.int32)
            for kr in range(8):
                gather_row(slab, g2 & 7, kr, colbase + kr)
            return ring_post(t, g2)

        return lax.fori_loop(t_lo, t_hi, body, g)

    def straddle(prev, cur, i, off_i, g):
        t0 = off_i // 8
        n1 = off_i - 8 * t0

        def do(g2):
            ring_pre(g2)
            par = g2 & 7

            pbase = jnp.full((16,), _N - n1, dtype=jnp.int32)
            cbase = jnp.full((16,), i - n1, dtype=jnp.int32)

            def kprev(kr, u):
                gather_row(prev, par, kr, pbase + kr)
                return u

            def kcur(kr, u):
                gather_row(cur, par, kr, cbase + kr)
                return u

            lax.fori_loop(0, n1, kprev, 0)
            lax.fori_loop(n1, 8, kcur, 0)
            return ring_post(t0, g2)

        return lax.cond((n1 != 0) & (t0 <= _TK - 3) & (i > 0),
                        do, lambda g2: g2, g)

    start_in(slab_a, 0)

    def pair(ip, g):
        i0 = ip * 2
        i1 = i0 + 1
        off0 = i0 * (2 * _N + 1 - i0) // 2
        off1 = off0 + (_N - i0)
        off2 = off1 + (_N - i1)

        wait_in(slab_a, i0)
        g = straddle(slab_b, slab_a, i0, off0, g)
        start_in(slab_b, i1)
        g = full_strips(slab_a, i0, off0, off1, g)

        wait_in(slab_b, i1)
        g = straddle(slab_a, slab_b, i1, off1, g)

        @pl.when(i1 + 1 < _N)
        def _():
            start_in(slab_a, i1 + 1)

        g = full_strips(slab_b, i1, off1, off2, g)
        return g

    g = lax.fori_loop(0, _N // 2, pair, 0)

    for j in range(6):
        pltpu.sync_copy(x_hbm.at[pl.ds(bc, _BPW), 122 + j, pl.ds(120, 8)],
                        tail_v.at[:, pl.ds(8 * j, 8)])

    for t, cols in _TAIL_TILES:
        ring_pre(g)
        par_t = g & 7
        for kr in range(8):
            colv = jnp.full((16,), cols[kr], dtype=jnp.int32)
            gather_row(tail_v, par_t, kr, colv)
        g = ring_post(t, g)

    for _ in range(8):
        drain_out()


def kernel(input):
    mesh = plsc.VectorSubcoreMesh(core_axis_name="c", subcore_axis_name="s")
    f = pl.kernel(
        _sc_body,
        mesh=mesh,
        out_type=jax.ShapeDtypeStruct((_TK, _NW, 8, _BPW), jnp.float32),
        scratch_types=[
            pltpu.VMEM((_BPW, _N), jnp.float32),
            pltpu.VMEM((_BPW, _N), jnp.float32),
            pltpu.VMEM((_BPW, 48), jnp.float32),
            pltpu.VMEM((8, 8, _BPW), jnp.float32),
            pltpu.SemaphoreType.DMA,
            pltpu.SemaphoreType.DMA,
        ],
        compiler_params=pltpu.CompilerParams(
            use_tc_tiling_on_sc=False, needs_layout_passes=False
        ),
    )
    r4 = f(input)
    return r4.transpose(1, 3, 0, 2).reshape(_B, _K)
